# baseline (speedup 1.0000x reference)
import functools

import jax
import jax.numpy as jnp
from jax import lax
from jax.experimental import pallas as pl
from jax.experimental.pallas import tpu as pltpu
from jax.experimental.pallas import tpu_sc as plsc

_CHUNK = 512


@functools.lru_cache(maxsize=None)
def _make_sc_gather(n_rows, d, b):
  info = plsc.get_sparse_core_info()
  nc, ns = info.num_cores, info.num_subcores
  nw = nc * ns
  assert b % nw == 0, (b, nw)
  b_per_w = b // nw
  main_cols = (n_rows // 128) * 128
  mesh = plsc.VectorSubcoreMesh(core_axis_name="c", subcore_axis_name="s")

  out_t = jax.ShapeDtypeStruct((b * d,), jnp.float32)

  @functools.partial(
      pl.kernel,
      out_type=(out_t, out_t, out_t, out_t),
      mesh=mesh,
      compiler_params=pltpu.CompilerParams(needs_layout_passes=False),
      scratch_types=[
          pltpu.VMEM((b_per_w + 32,), jnp.int32),
          pltpu.VMEM((b_per_w + 32,), jnp.int32),
          pltpu.VMEM((d, 2 * _CHUNK), jnp.float32),
          pltpu.VMEM((b_per_w * d,), jnp.float32),
          pltpu.SemaphoreType.DMA,
          pltpu.SemaphoreType.DMA,
          pltpu.SemaphoreType.DMA,
      ],
  )
  def sc_gather(su_hbm, pu_hbm, si_hbm, pi_hbm,
                ugt_hbm, igt_hbm, umt_hbm, imt_hbm,
                ugtail_hbm, igtail_hbm, umtail_hbm, imtail_hbm,
                out_ug, out_ig, out_um, out_im,
                svals, spos, stage, outbuf, wsem, tsem, psem):
    wid = lax.axis_index("s") * nc + lax.axis_index("c")
    base = wid * b_per_w
    last_chunk = main_cols - _CHUNK
    row_iotas = [
        lax.iota(jnp.int32, 16) + (q * 16) for q in range(d // 16)
    ]

    def run_table(table, tail, out, sv_hbm, sp_hbm):
      pltpu.sync_copy(sv_hbm.at[pl.ds(base, b_per_w)],
                      svals.at[pl.ds(0, b_per_w)])
      pltpu.sync_copy(sp_hbm.at[pl.ds(base, b_per_w)],
                      spos.at[pl.ds(0, b_per_w)])

      def slot_off(c):
        return pl.multiple_of(((c // _CHUNK) % 2) * _CHUNK, 128)

      def stage_chunk(c, sem):
        return pltpu.async_copy(
            table.at[:, pl.ds(pl.multiple_of(c, 128), _CHUNK)],
            stage.at[:, pl.ds(slot_off(c), _CHUNK)], sem)

      def prefetch_after(c):
        stage_chunk(jnp.minimum(c + _CHUNK, last_chunk), psem)

      def emit(k, pos):
        pltpu.async_copy(outbuf.at[pl.ds(k * d, d)],
                         out.at[pl.ds(pos * d, d)], wsem)

      def select(col, k):
        for q, rows in enumerate(row_iotas):
          vals = plsc.load_gather(
              stage, [rows, jnp.broadcast_to(col, (16,))])
          outbuf[pl.ds(k * d + q * 16, 16)] = vals

      val0 = svals[pl.ds(0, 16)][0]
      clo0 = jnp.minimum((val0 // _CHUNK) * _CHUNK, last_chunk)
      stage_chunk(clo0, tsem).wait()
      prefetch_after(clo0)

      def entry(k, val, pos, clo):
        def main_fn(c):
          need = (val // _CHUNK) * _CHUNK

          def cross_fn(c2):
            del c2
            pltpu.make_async_copy(
                table.at[:, pl.ds(0, _CHUNK)],
                stage.at[:, pl.ds(0, _CHUNK)], psem).wait()

            def miss_fn(_):
              stage_chunk(need, tsem).wait()
              return 0

            lax.cond(need != clo + _CHUNK, miss_fn, lambda _: 0, 0)
            prefetch_after(need)
            return need

          c2 = lax.cond(need != c, cross_fn, lambda c2: c2, c)
          select(slot_off(need) + (val - need), k)
          emit(k, pos)
          return c2

        def tail_fn(c):
          pltpu.async_copy(tail.at[pl.ds((val - main_cols) * d, d)],
                           outbuf.at[pl.ds(k * d, d)], tsem).wait()
          emit(k, pos)
          return c

        return lax.cond(val < main_cols, main_fn, tail_fn, clo)

      def group(g, clo):
        vals = svals[pl.ds(g * 16, 16)]
        poss = spos[pl.ds(g * 16, 16)]
        for j in range(16):
          clo = entry(g * 16 + j, vals[j], poss[j], clo)
        return clo

      lax.fori_loop(0, b_per_w // 16, group, clo0)

      pltpu.make_async_copy(
          table.at[:, pl.ds(0, _CHUNK)],
          stage.at[:, pl.ds(0, _CHUNK)], psem).wait()
      pltpu.make_async_copy(out.at[pl.ds(0, b_per_w * d)], outbuf,
                            wsem).wait()

    run_table(ugt_hbm, ugtail_hbm, out_ug, su_hbm, pu_hbm)
    run_table(umt_hbm, umtail_hbm, out_um, su_hbm, pu_hbm)
    run_table(igt_hbm, igtail_hbm, out_ig, si_hbm, pi_hbm)
    run_table(imt_hbm, imtail_hbm, out_im, si_hbm, pi_hbm)

  return sc_gather


def _mlp_body(gu, gi, mu, mi, w0a, w0b, b0, w1, b1, w2, b2, wog, woh, bo,
              out):
  f32 = jnp.float32
  pr = lax.Precision.HIGHEST
  x = jnp.dot(mu[...], w0a[...], preferred_element_type=f32, precision=pr)
  x = x + jnp.dot(mi[...], w0b[...], preferred_element_type=f32,
                  precision=pr)
  h = jnp.maximum(x + b0[...], 0.0)
  h = jnp.maximum(
      jnp.dot(h, w1[...], preferred_element_type=f32, precision=pr)
      + b1[...], 0.0)
  h = jnp.maximum(
      jnp.dot(h, w2[...], preferred_element_type=f32, precision=pr)
      + b2[...], 0.0)
  g = gu[...] * gi[...]
  acc = (jnp.sum(g * wog[...], axis=1, keepdims=True)
         + jnp.sum(h * woh[...], axis=1, keepdims=True) + bo[...])
  out[...] = acc


def _mlp_forward(gu, gi, mu, mi, W0, b0, W1, b1, W2, b2, Wo, bo,
                 block_rows=2048, interpret=False):
  b, d = gu.shape
  d0 = W0.shape[1]
  d1 = W1.shape[1]
  d2 = W2.shape[1]
  assert b % block_rows == 0
  w0a = W0[:d]
  w0b = W0[d:]
  wog = Wo[:d, 0].reshape(1, d)
  woh = Wo[d:, 0].reshape(1, d2)
  row = lambda i: (i, 0)
  fixed = lambda i: (0, 0)
  out = pl.pallas_call(
      _mlp_body,
      grid=(b // block_rows,),
      in_specs=[
          pl.BlockSpec((block_rows, d), row),
          pl.BlockSpec((block_rows, d), row),
          pl.BlockSpec((block_rows, d), row),
          pl.BlockSpec((block_rows, d), row),
          pl.BlockSpec((d, d0), fixed),
          pl.BlockSpec((d, d0), fixed),
          pl.BlockSpec((1, d0), fixed),
          pl.BlockSpec((d0, d1), fixed),
          pl.BlockSpec((1, d1), fixed),
          pl.BlockSpec((d1, d2), fixed),
          pl.BlockSpec((1, d2), fixed),
          pl.BlockSpec((1, d), fixed),
          pl.BlockSpec((1, d2), fixed),
          pl.BlockSpec((1, 1), fixed),
      ],
      out_specs=pl.BlockSpec((block_rows, 1), row),
      out_shape=jax.ShapeDtypeStruct((b, 1), jnp.float32),
      interpret=interpret,
  )(gu, gi, mu, mi, w0a, w0b, b0.reshape(1, d0), W1, b1.reshape(1, d1),
    W2, b2.reshape(1, d2), wog, woh, bo.reshape(1, 1))
  return out[:, 0]


def kernel(user_indices, item_indices, user_emb_gmf, item_emb_gmf,
           user_emb_mlp, item_emb_mlp, W0, b0, W1, b1, W2, b2, Wo, bo):
  b = user_indices.shape[0]
  nu, d = user_emb_gmf.shape
  u = jnp.clip(user_indices, 0, nu - 1)
  i = jnp.clip(item_indices, 0, nu - 1)
  pu = jnp.argsort(u).astype(jnp.int32)
  su = u[pu]
  pi_ = jnp.argsort(i).astype(jnp.int32)
  si = i[pi_]
  main_cols = (nu // 128) * 128
  tails = [t[main_cols:].reshape(-1)
           for t in (user_emb_gmf, item_emb_gmf, user_emb_mlp,
                     item_emb_mlp)]
  sc_gather = _make_sc_gather(nu, d, b)
  fug, fig, fum, fim = sc_gather(
      su, pu, si, pi_, user_emb_gmf.T, item_emb_gmf.T, user_emb_mlp.T,
      item_emb_mlp.T, *tails)
  gu = fug.reshape(b, d)
  gi = fig.reshape(b, d)
  mu = fum.reshape(b, d)
  mi = fim.reshape(b, d)
  return _mlp_forward(gu, gi, mu, mi, W0, b0, W1, b1, W2, b2, Wo, bo)


# speedup vs baseline: 1.0603x; 1.0603x over previous
import functools

import jax
import jax.numpy as jnp
from jax import lax
from jax.experimental import pallas as pl
from jax.experimental.pallas import tpu as pltpu
from jax.experimental.pallas import tpu_sc as plsc

_CHUNK = 512


@functools.lru_cache(maxsize=None)
def _make_sc_gather(n_rows, d, b):
  info = plsc.get_sparse_core_info()
  nc, ns = info.num_cores, info.num_subcores
  nw = nc * ns
  assert b % nw == 0, (b, nw)
  b_per_w = b // nw
  main_cols = (n_rows // 128) * 128
  mesh = plsc.VectorSubcoreMesh(core_axis_name="c", subcore_axis_name="s")

  out_t = jax.ShapeDtypeStruct((b * d,), jnp.float32)

  @functools.partial(
      pl.kernel,
      out_type=(out_t, out_t, out_t, out_t),
      mesh=mesh,
      compiler_params=pltpu.CompilerParams(needs_layout_passes=False),
      scratch_types=[
          pltpu.VMEM((b_per_w + 32,), jnp.int32),
          pltpu.VMEM((b_per_w + 32,), jnp.int32),
          pltpu.VMEM((d, 2 * _CHUNK), jnp.float32),
          pltpu.VMEM((b_per_w * d,), jnp.float32),
          pltpu.SemaphoreType.DMA,
          pltpu.SemaphoreType.DMA,
          pltpu.SemaphoreType.DMA,
      ],
  )
  def sc_gather(su_hbm, pu_hbm, si_hbm, pi_hbm,
                ugt_hbm, igt_hbm, umt_hbm, imt_hbm,
                ugtail_hbm, igtail_hbm, umtail_hbm, imtail_hbm,
                out_ug, out_ig, out_um, out_im,
                svals, spos, stage, outbuf, wsem, tsem, psem):
    wid = lax.axis_index("s") * nc + lax.axis_index("c")
    base = wid * b_per_w
    last_chunk = main_cols - _CHUNK
    row_iotas = [
        lax.iota(jnp.int32, 16) + (q * 16) for q in range(d // 16)
    ]

    def run_table(table, tail, out, sv_hbm, sp_hbm):
      pltpu.sync_copy(sv_hbm.at[pl.ds(base, b_per_w)],
                      svals.at[pl.ds(0, b_per_w)])
      pltpu.sync_copy(sp_hbm.at[pl.ds(base, b_per_w)],
                      spos.at[pl.ds(0, b_per_w)])

      def slot_off(c):
        return pl.multiple_of(((c // _CHUNK) % 2) * _CHUNK, 128)

      def stage_chunk(c, sem):
        return pltpu.async_copy(
            table.at[:, pl.ds(pl.multiple_of(c, 128), _CHUNK)],
            stage.at[:, pl.ds(slot_off(c), _CHUNK)], sem)

      def prefetch_after(c):
        stage_chunk(jnp.minimum(c + _CHUNK, last_chunk), psem)

      def emit(k, pos):
        pltpu.async_copy(outbuf.at[pl.ds(k * d, d)],
                         out.at[pl.ds(pos * d, d)], wsem)

      def select(col, k):
        for q, rows in enumerate(row_iotas):
          vals = plsc.load_gather(
              stage, [rows, jnp.broadcast_to(col, (16,))])
          outbuf[pl.ds(k * d + q * 16, 16)] = vals

      val0 = svals[pl.ds(0, 16)][0]
      clo0 = jnp.minimum((val0 // _CHUNK) * _CHUNK, last_chunk)
      stage_chunk(clo0, tsem).wait()
      prefetch_after(clo0)

      def entry(k, val, pos, clo):
        def main_fn(c):
          need = (val // _CHUNK) * _CHUNK

          def cross_fn(c2):
            del c2
            pltpu.make_async_copy(
                table.at[:, pl.ds(0, _CHUNK)],
                stage.at[:, pl.ds(0, _CHUNK)], psem).wait()

            def miss_fn(_):
              stage_chunk(need, tsem).wait()
              return 0

            lax.cond(need != clo + _CHUNK, miss_fn, lambda _: 0, 0)
            prefetch_after(need)
            return need

          c2 = lax.cond(need != c, cross_fn, lambda c2: c2, c)
          select(slot_off(need) + (val - need), k)
          emit(k, pos)
          return c2

        def tail_fn(c):
          pltpu.async_copy(tail.at[pl.ds((val - main_cols) * d, d)],
                           outbuf.at[pl.ds(k * d, d)], tsem).wait()
          emit(k, pos)
          return c

        return lax.cond(val < main_cols, main_fn, tail_fn, clo)

      def group(g, clo):
        vals = svals[pl.ds(g * 16, 16)]
        poss = spos[pl.ds(g * 16, 16)]
        for j in range(16):
          clo = entry(g * 16 + j, vals[j], poss[j], clo)
        return clo

      lax.fori_loop(0, b_per_w // 16, group, clo0)

      pltpu.make_async_copy(
          table.at[:, pl.ds(0, _CHUNK)],
          stage.at[:, pl.ds(0, _CHUNK)], psem).wait()
      pltpu.make_async_copy(out.at[pl.ds(0, b_per_w * d)], outbuf,
                            wsem).wait()

    run_table(ugt_hbm, ugtail_hbm, out_ug, su_hbm, pu_hbm)
    run_table(umt_hbm, umtail_hbm, out_um, su_hbm, pu_hbm)
    run_table(igt_hbm, igtail_hbm, out_ig, si_hbm, pi_hbm)
    run_table(imt_hbm, imtail_hbm, out_im, si_hbm, pi_hbm)

  return sc_gather


def _mlp_body(gu, gi, mu, mi, w0a, w0b, b0, w1, b1, w2, b2, wog, woh, bo,
              out):
  f32 = jnp.float32
  bf16 = jnp.bfloat16
  pr = lax.Precision.DEFAULT
  x = jnp.dot(mu[...].astype(bf16), w0a[...].astype(bf16),
              preferred_element_type=f32, precision=pr)
  x = x + jnp.dot(mi[...].astype(bf16), w0b[...].astype(bf16),
                  preferred_element_type=f32, precision=pr)
  h = jnp.maximum(x + b0[...], 0.0)
  h = jnp.maximum(
      jnp.dot(h.astype(bf16), w1[...].astype(bf16),
              preferred_element_type=f32, precision=pr) + b1[...], 0.0)
  h = jnp.maximum(
      jnp.dot(h.astype(bf16), w2[...].astype(bf16),
              preferred_element_type=f32, precision=pr) + b2[...], 0.0)
  g = gu[...] * gi[...]
  acc = (jnp.sum(g * wog[...], axis=1, keepdims=True)
         + jnp.sum(h * woh[...], axis=1, keepdims=True) + bo[...])
  out[...] = acc


def _mlp_forward(gu, gi, mu, mi, W0, b0, W1, b1, W2, b2, Wo, bo,
                 block_rows=2048, interpret=False):
  b, d = gu.shape
  d0 = W0.shape[1]
  d1 = W1.shape[1]
  d2 = W2.shape[1]
  assert b % block_rows == 0
  w0a = W0[:d]
  w0b = W0[d:]
  wog = Wo[:d, 0].reshape(1, d)
  woh = Wo[d:, 0].reshape(1, d2)
  row = lambda i: (i, 0)
  fixed = lambda i: (0, 0)
  out = pl.pallas_call(
      _mlp_body,
      grid=(b // block_rows,),
      in_specs=[
          pl.BlockSpec((block_rows, d), row),
          pl.BlockSpec((block_rows, d), row),
          pl.BlockSpec((block_rows, d), row),
          pl.BlockSpec((block_rows, d), row),
          pl.BlockSpec((d, d0), fixed),
          pl.BlockSpec((d, d0), fixed),
          pl.BlockSpec((1, d0), fixed),
          pl.BlockSpec((d0, d1), fixed),
          pl.BlockSpec((1, d1), fixed),
          pl.BlockSpec((d1, d2), fixed),
          pl.BlockSpec((1, d2), fixed),
          pl.BlockSpec((1, d), fixed),
          pl.BlockSpec((1, d2), fixed),
          pl.BlockSpec((1, 1), fixed),
      ],
      out_specs=pl.BlockSpec((block_rows, 1), row),
      out_shape=jax.ShapeDtypeStruct((b, 1), jnp.float32),
      interpret=interpret,
  )(gu, gi, mu, mi, w0a, w0b, b0.reshape(1, d0), W1, b1.reshape(1, d1),
    W2, b2.reshape(1, d2), wog, woh, bo.reshape(1, 1))
  return out[:, 0]


def kernel(user_indices, item_indices, user_emb_gmf, item_emb_gmf,
           user_emb_mlp, item_emb_mlp, W0, b0, W1, b1, W2, b2, Wo, bo):
  b = user_indices.shape[0]
  nu, d = user_emb_gmf.shape
  u = jnp.clip(user_indices, 0, nu - 1)
  i = jnp.clip(item_indices, 0, nu - 1)
  pu = jnp.argsort(u).astype(jnp.int32)
  su = u[pu]
  pi_ = jnp.argsort(i).astype(jnp.int32)
  si = i[pi_]
  main_cols = (nu // 128) * 128
  tails = [t[main_cols:].reshape(-1)
           for t in (user_emb_gmf, item_emb_gmf, user_emb_mlp,
                     item_emb_mlp)]
  sc_gather = _make_sc_gather(nu, d, b)
  fug, fig, fum, fim = sc_gather(
      su, pu, si, pi_, user_emb_gmf.T, item_emb_gmf.T, user_emb_mlp.T,
      item_emb_mlp.T, *tails)
  gu = fug.reshape(b, d)
  gi = fig.reshape(b, d)
  mu = fum.reshape(b, d)
  mi = fim.reshape(b, d)
  return _mlp_forward(gu, gi, mu, mi, W0, b0, W1, b1, W2, b2, Wo, bo)
